# baseline (speedup 1.0000x reference)
import jax
import jax.numpy as jnp
from jax.experimental import pallas as pl
from jax.experimental.pallas import tpu as pltpu

T_DIM = 4096
S_DIM = 2048
B_DIM = 1024
N_TOK = 16
N_EXP = 8
TOP_K = 2
G_DIM = 512
N_HEADS = 8
HD = S_DIM // N_HEADS
SEQ = 2048
BATCH = 2

BS = 256
NS = SEQ // BS

_DN = (((1,), (1,)), ((), ()))

f32 = jnp.float32
bf16 = jnp.bfloat16


def _gelu_exact(x):
    return 0.5 * x * (1.0 + jax.lax.erf(x * 0.7071067811865476))


def _ln_f32(x, g, b, eps=1e-5):
    mu = jnp.mean(x, axis=-1, keepdims=True)
    var = jnp.mean((x - mu) ** 2, axis=-1, keepdims=True)
    return (x - mu) / jnp.sqrt(var + eps) * g + b


RBS = 256
NSR = SEQ // RBS


def _router_kernel(h_ref, rw1_ref, rb1_ref, rw2_ref, rb2_ref,
                   probs_ref, w_ref, idx_ref, psum_scr):
    s = pl.program_id(1)

    @pl.when(s == 0)
    def _init():
        psum_scr[...] = jnp.zeros((1, T_DIM), f32)

    psum_scr[...] += jnp.sum(h_ref[0], axis=0, keepdims=True)

    @pl.when(s == NSR - 1)
    def _finish():
        pooled = psum_scr[...] / SEQ
        hid = jax.lax.dot_general(pooled, rw1_ref[...], _DN,
                                  preferred_element_type=f32) + rb1_ref[...]
        hid = _gelu_exact(hid)
        logits = jax.lax.dot_general(hid, rw2_ref[...], _DN,
                                     preferred_element_type=f32) + rb2_ref[...]
        z = logits - jnp.max(logits, axis=-1, keepdims=True)
        ez = jnp.exp(z)
        probs = ez / jnp.sum(ez, axis=-1, keepdims=True)
        probs_ref[0] = probs

        ids = jax.lax.broadcasted_iota(jnp.int32, (1, N_EXP), 1)
        m1 = jnp.max(probs)
        i1 = jnp.min(jnp.where(probs == m1, ids, N_EXP))
        probs2 = jnp.where(ids == i1, -jnp.inf, probs)
        m2 = jnp.max(probs2)
        i2 = jnp.min(jnp.where(probs2 == m2, ids, N_EXP))
        denom = m1 + m2 + 1e-8
        pick = jax.lax.broadcasted_iota(jnp.int32, (1, TOP_K), 1)
        w_ref[0] = jnp.where(pick == 0, m1, m2) / denom
        idx_ref[0] = jnp.where(pick == 0, i1, i2).astype(jnp.int32)


def _run_router(h, rW1, rb1, rW2, rb2):
    probs, w, idx = pl.pallas_call(
        _router_kernel,
        grid=(BATCH, NSR),
        in_specs=[
            pl.BlockSpec((1, RBS, T_DIM), lambda b, s: (b, s, 0)),
            pl.BlockSpec((G_DIM, T_DIM), lambda b, s: (0, 0)),
            pl.BlockSpec((1, G_DIM), lambda b, s: (0, 0)),
            pl.BlockSpec((N_EXP, G_DIM), lambda b, s: (0, 0)),
            pl.BlockSpec((1, N_EXP), lambda b, s: (0, 0)),
        ],
        out_specs=[
            pl.BlockSpec((1, 1, N_EXP), lambda b, s: (b, 0, 0)),
            pl.BlockSpec((1, 1, TOP_K), lambda b, s: (b, 0, 0)),
            pl.BlockSpec((1, 1, TOP_K), lambda b, s: (b, 0, 0)),
        ],
        out_shape=[
            jax.ShapeDtypeStruct((BATCH, 1, N_EXP), f32),
            jax.ShapeDtypeStruct((BATCH, 1, TOP_K), f32),
            jax.ShapeDtypeStruct((BATCH, 1, TOP_K), jnp.int32),
        ],
        scratch_shapes=[pltpu.VMEM((1, T_DIM), f32)],
        compiler_params=pltpu.CompilerParams(
            dimension_semantics=("arbitrary", "arbitrary")),
    )(h, rW1, rb1.reshape(1, G_DIM), rW2, rb2.reshape(1, N_EXP))
    return probs.reshape(BATCH, N_EXP), w.reshape(BATCH, TOP_K), \
        idx.reshape(BATCH, TOP_K)


def _p_kernel(q_ref, bq_ref, wq_ref, wk_ref, p_ref):
    qq = jax.lax.dot_general(q_ref[...].astype(bf16),
                             wq_ref[...].astype(bf16), _DN,
                             preferred_element_type=f32) + bq_ref[...]
    rows = []
    for h_i in range(N_HEADS):
        sl = slice(h_i * HD, (h_i + 1) * HD)
        rows.append(jnp.dot(qq[:, sl].astype(bf16),
                            wk_ref[sl, :].astype(bf16),
                            preferred_element_type=f32))
    p_ref[...] = jnp.concatenate(rows, axis=0).astype(bf16)


def _run_p(q, bq, Win):
    return pl.pallas_call(
        _p_kernel,
        grid=(1,),
        in_specs=[
            pl.BlockSpec((N_TOK, S_DIM), lambda i: (0, 0)),
            pl.BlockSpec((1, S_DIM), lambda i: (0, 0)),
            pl.BlockSpec((S_DIM, S_DIM), lambda i: (0, 0)),
            pl.BlockSpec((S_DIM, S_DIM), lambda i: (1, 0)),
        ],
        out_specs=pl.BlockSpec((N_HEADS * N_TOK, S_DIM), lambda i: (0, 0)),
        out_shape=jax.ShapeDtypeStruct((N_HEADS * N_TOK, S_DIM), bf16),
        compiler_params=pltpu.CompilerParams(
            dimension_semantics=("arbitrary",)),
    )(q, bq, Win, Win)


def _mlp_kernel(h_ref, wd_ref, bd_ref, wu_ref, bu_ref, lng_ref, lnb_ref,
                x2_ref):
    hb = h_ref[0].astype(bf16)
    x1 = jax.lax.dot_general(hb, wd_ref[...], _DN,
                             preferred_element_type=f32) + bd_ref[...]
    x1 = _gelu_exact(x1)
    x2 = jax.lax.dot_general(x1.astype(bf16), wu_ref[...], _DN,
                             preferred_element_type=f32) + bu_ref[...]
    x2 = _ln_f32(x2, lng_ref[...], lnb_ref[...])
    x2_ref[0] = x2.astype(bf16)


def _run_mlp(h, Wd, bd, Wu, bu, lng, lnb, batches):
    nb = len(batches)

    def h_map(b, s, _bt=tuple(batches)):
        if len(_bt) == 1:
            return (_bt[0], s, 0)
        return (b, s, 0)

    return pl.pallas_call(
        _mlp_kernel,
        grid=(nb, NS),
        in_specs=[
            pl.BlockSpec((1, BS, T_DIM), h_map),
            pl.BlockSpec((B_DIM, T_DIM), lambda b, s: (0, 0)),
            pl.BlockSpec((1, B_DIM), lambda b, s: (0, 0)),
            pl.BlockSpec((S_DIM, B_DIM), lambda b, s: (0, 0)),
            pl.BlockSpec((1, S_DIM), lambda b, s: (0, 0)),
            pl.BlockSpec((1, S_DIM), lambda b, s: (0, 0)),
            pl.BlockSpec((1, S_DIM), lambda b, s: (0, 0)),
        ],
        out_specs=pl.BlockSpec((1, BS, S_DIM), lambda b, s: (b, s, 0)),
        out_shape=jax.ShapeDtypeStruct((nb, SEQ, S_DIM), bf16),
        compiler_params=pltpu.CompilerParams(
            dimension_semantics=("arbitrary", "arbitrary")),
    )(h, Wd, bd, Wu, bu, lng, lnb)


def _attn_kernel(x2_ref, p_ref, wv_ref, bv_ref, wo_ref, bo_ref, q_ref,
                 png_ref, pnb_ref, wsc_ref, val_ref, sc_scr, vv_scr):
    s = pl.program_id(1)
    x2c = x2_ref[0]
    vv = jax.lax.dot_general(x2c, wv_ref[...], _DN,
                             preferred_element_type=f32) + bv_ref[...]
    vv_scr[pl.ds(s * BS, BS), :] = vv.astype(bf16)
    sc_scr[:, pl.ds(s * BS, BS)] = jax.lax.dot_general(
        p_ref[...], x2c, _DN, preferred_element_type=f32)

    @pl.when(s == NS - 1)
    def _attention():
        ctx_heads = []
        for h_i in range(N_HEADS):
            sc = sc_scr[pl.ds(h_i * N_TOK, N_TOK), :] / 16.0
            att = jax.nn.softmax(sc, axis=-1)
            vh = vv_scr[:, h_i * HD:(h_i + 1) * HD]
            ctx_heads.append(
                jnp.dot(att.astype(bf16), vh, preferred_element_type=f32))
        ctx = jnp.concatenate(ctx_heads, axis=-1)
        val = jax.lax.dot_general(ctx.astype(bf16), wo_ref[...], _DN,
                                  preferred_element_type=f32) + bo_ref[...]
        val = _ln_f32(val + q_ref[...], png_ref[...], pnb_ref[...])
        val_ref[0] = val * wsc_ref[0, 0]


def _run_attn(x2, P, Wv, bv, Wo, bo, q, png, pnb, wscale):
    nb = x2.shape[0]
    return pl.pallas_call(
        _attn_kernel,
        grid=(nb, NS),
        in_specs=[
            pl.BlockSpec((1, BS, S_DIM), lambda b, s: (b, s, 0)),
            pl.BlockSpec((N_HEADS * N_TOK, S_DIM), lambda b, s: (0, 0)),
            pl.BlockSpec((S_DIM, S_DIM), lambda b, s: (0, 0)),
            pl.BlockSpec((1, S_DIM), lambda b, s: (0, 0)),
            pl.BlockSpec((S_DIM, S_DIM), lambda b, s: (0, 0)),
            pl.BlockSpec((1, S_DIM), lambda b, s: (0, 0)),
            pl.BlockSpec((N_TOK, S_DIM), lambda b, s: (0, 0)),
            pl.BlockSpec((1, S_DIM), lambda b, s: (0, 0)),
            pl.BlockSpec((1, S_DIM), lambda b, s: (0, 0)),
            pl.BlockSpec((1, 1), lambda b, s: (0, 0)),
        ],
        out_specs=pl.BlockSpec((1, N_TOK, S_DIM), lambda b, s: (b, 0, 0)),
        out_shape=jax.ShapeDtypeStruct((nb, N_TOK, S_DIM), f32),
        scratch_shapes=[
            pltpu.VMEM((N_HEADS * N_TOK, SEQ), f32),
            pltpu.VMEM((SEQ, S_DIM), bf16),
        ],
        compiler_params=pltpu.CompilerParams(
            dimension_semantics=("arbitrary", "arbitrary")),
    )(x2, P, Wv, bv, Wo, bo, q, png, pnb, wscale)


def _pair_pipeline(h, p_, batches, wscale):
    P = _run_p(p_['q'][0], p_['bin'][None, :S_DIM], p_['Win'])
    x2 = _run_mlp(h, p_['Wd'].astype(bf16), p_['bd'][None],
                  p_['Wu'].astype(bf16), p_['bu'][None],
                  p_['ln_g'][None], p_['ln_b'][None], batches)
    return _run_attn(x2, P,
                     p_['Win'][2 * S_DIM:].astype(bf16),
                     p_['bin'][None, 2 * S_DIM:],
                     p_['Wo'].astype(bf16), p_['bo'][None],
                     p_['q'][0], p_['pn_g'][None], p_['pn_b'][None],
                     wscale)


def kernel(h_teacher, params):
    probs, w, idx = _run_router(h_teacher, params['rW1'], params['rb1'],
                                params['rW2'], params['rb2'])

    one = jnp.ones((1, 1), f32)
    c_g = _pair_pipeline(h_teacher, params['gen'], (0, 1), one)

    flat_idx = idx.reshape(2 * TOP_K)
    flat_w = w.reshape(2 * TOP_K)
    vals = []
    for j, b_static in enumerate((0, 0, 1, 1)):
        branches = [
            (lambda wj, p_=p_, b_=b_static:
             _pair_pipeline(h_teacher, p_, (b_,), wj))
            for p_ in params['spec']
        ]
        vals.append(jax.lax.switch(flat_idx[j], branches,
                                   flat_w[j].reshape(1, 1)))
    comb = jnp.concatenate([vals[0] + vals[1], vals[2] + vals[3]], axis=0)
    c_agg = jnp.concatenate([c_g, comb], axis=1)
    return c_agg, probs
